# Initial kernel scaffold; baseline (speedup 1.0000x reference)
import functools

import jax
import jax.numpy as jnp
from jax import lax
from jax.experimental import pallas as pl
from jax.experimental.pallas import tpu as pltpu
from jax.experimental.pallas import tpu_sc as plsc

_N = 100000
_D = 128
_G = 256
_B = 80
_NB = _N // _B
_NC = 2
_NS = 16
_NW = _NC * _NS
_BLK_LO = _NB // _NW
_EXTRA = _NB - _BLK_LO * _NW

_mesh = plsc.VectorSubcoreMesh(core_axis_name="c", subcore_axis_name="s")


def _worker_range(wid):
    start = wid * _BLK_LO + jnp.minimum(wid, _EXTRA)
    cnt = _BLK_LO + jnp.where(wid < _EXTRA, 1, 0)
    return start, start + cnt


def _zero_rows(ref, nrows, ncols):
    z = jnp.zeros((16,), jnp.float32)
    for i in range(nrows):
        for j in range(ncols // 16):
            ref[i, pl.ds(16 * j, 16)] = z


@functools.partial(
    pl.kernel,
    out_type=(
        jax.ShapeDtypeStruct((2 * _G, _D), jnp.float32),
        jax.ShapeDtypeStruct((2 * _G, 16), jnp.float32),
    ),
    mesh=_mesh,
    scratch_types=[
        pltpu.VMEM((_B, _D), jnp.float32),
        pltpu.VMEM((_B,), jnp.int32),
        pltpu.VMEM((_B, 16), jnp.float32),
        pltpu.VMEM((16, _D), jnp.float32),
        pltpu.VMEM((16, 16), jnp.float32),
        pltpu.VMEM_SHARED((_G, _D), jnp.float32),
        pltpu.VMEM_SHARED((_G, 16), jnp.float32),
    ],
)
def _sc_pass1(x_hbm, b_hbm, sums_out, cnts_out, xb, bb, ones, tmp, tmpc, ssum, scnt):
    cid = lax.axis_index("c")
    sid = lax.axis_index("s")
    wid = cid * _NS + sid

    one = jnp.ones((16,), jnp.float32)
    for r in range(_B):
        ones[r, :] = one

    _zero_rows(tmp, 16, _D)
    _zero_rows(tmpc, 16, 16)
    pltpu.sync_copy(tmp, ssum.at[pl.ds(sid * 16, 16)])
    pltpu.sync_copy(tmpc, scnt.at[pl.ds(sid * 16, 16)])
    plsc.subcore_barrier()

    start, end = _worker_range(wid)

    def blk_body(blk, carry):
        base = pl.multiple_of(blk * _B, 16)
        pltpu.sync_copy(x_hbm.at[pl.ds(base, _B)], xb)
        pltpu.sync_copy(b_hbm.at[pl.ds(base, _B)], bb)
        pltpu.sync_copy(xb, ssum.at[bb], add=True)
        pltpu.sync_copy(ones, scnt.at[bb], add=True)
        return carry

    lax.fori_loop(start, end, blk_body, 0)
    plsc.subcore_barrier()

    pltpu.sync_copy(ssum.at[pl.ds(sid * 16, 16)], tmp)
    pltpu.sync_copy(tmp, sums_out.at[pl.ds(cid * _G + sid * 16, 16)])
    pltpu.sync_copy(scnt.at[pl.ds(sid * 16, 16)], tmpc)
    pltpu.sync_copy(tmpc, cnts_out.at[pl.ds(cid * _G + sid * 16, 16)])


@functools.partial(
    pl.kernel,
    out_type=jax.ShapeDtypeStruct((2 * _G, _D), jnp.float32),
    mesh=_mesh,
    scratch_types=[
        pltpu.VMEM((_B, _D), jnp.float32),
        pltpu.VMEM((_B,), jnp.int32),
        pltpu.VMEM((_G, _D), jnp.float32),
        pltpu.VMEM((_B,), jnp.float32),
        pltpu.VMEM((16, _D), jnp.float32),
        pltpu.VMEM_SHARED((_G, _D), jnp.float32),
    ],
)
def _sc_pass2(x_hbm, b_hbm, tg_hbm, out2, xb, bb, tgv, dots, tmp, sout):
    cid = lax.axis_index("c")
    sid = lax.axis_index("s")
    wid = cid * _NS + sid

    pltpu.sync_copy(tg_hbm, tgv)
    _zero_rows(tmp, 16, _D)
    pltpu.sync_copy(tmp, sout.at[pl.ds(sid * 16, 16)])
    plsc.subcore_barrier()

    start, end = _worker_range(wid)

    def blk_body(blk, carry):
        base = pl.multiple_of(blk * _B, 16)
        pltpu.sync_copy(x_hbm.at[pl.ds(base, _B)], xb)
        pltpu.sync_copy(b_hbm.at[pl.ds(base, _B)], bb)

        def row_dot(r, c2):
            b = bb[r]
            acc = xb[r, pl.ds(0, 16)] * tgv[b, pl.ds(0, 16)]
            for j in range(1, _D // 16):
                acc = acc + xb[r, pl.ds(16 * j, 16)] * tgv[b, pl.ds(16 * j, 16)]
            dots[r] = jnp.sum(acc)
            return c2

        lax.fori_loop(0, _B, row_dot, 0)

        for k in range(_B // 16):
            d = dots[pl.ds(16 * k, 16)]
            dots[pl.ds(16 * k, 16)] = 1.0 / (1.0 + jnp.exp(-d))

        def row_scale(r, c2):
            c = dots[r]
            for j in range(_D // 16):
                xb[r, pl.ds(16 * j, 16)] = xb[r, pl.ds(16 * j, 16)] * c
            return c2

        lax.fori_loop(0, _B, row_scale, 0)
        pltpu.sync_copy(xb, sout.at[bb], add=True)
        return carry

    lax.fori_loop(start, end, blk_body, 0)
    plsc.subcore_barrier()

    pltpu.sync_copy(sout.at[pl.ds(sid * 16, 16)], tmp)
    pltpu.sync_copy(tmp, out2.at[pl.ds(cid * _G + sid * 16, 16)])


def _tc_mid(sums2, cnts2, W):
    def body(s_ref, c_ref, w_ref, o_ref):
        s = s_ref[pl.ds(0, _G), :] + s_ref[pl.ds(_G, _G), :]
        c = c_ref[pl.ds(0, _G), :][:, 0:1] + c_ref[pl.ds(_G, _G), :][:, 0:1]
        mean = s / jnp.maximum(c, 1.0)
        o_ref[...] = jnp.tanh(
            jnp.dot(mean, w_ref[...], preferred_element_type=jnp.float32))

    return pl.pallas_call(
        body, out_shape=jax.ShapeDtypeStruct((_G, _D), jnp.float32))(sums2, cnts2, W)


def _tc_combine(p2):
    def body(p_ref, o_ref):
        o_ref[...] = p_ref[pl.ds(0, _G), :] + p_ref[pl.ds(_G, _G), :]

    return pl.pallas_call(
        body, out_shape=jax.ShapeDtypeStruct((_G, _D), jnp.float32))(p2)


def kernel(x, batch, W):
    b32 = batch.astype(jnp.int32)
    sums2, cnts2 = _sc_pass1(x, b32)
    tg = _tc_mid(sums2, cnts2, W)
    p2 = _sc_pass2(x, b32, tg)
    return _tc_combine(p2)


# speedup vs baseline: 1.6244x; 1.6244x over previous
import functools

import jax
import jax.numpy as jnp
from jax import lax
from jax.experimental import pallas as pl
from jax.experimental.pallas import tpu as pltpu
from jax.experimental.pallas import tpu_sc as plsc

_N = 100000
_D = 128
_G = 256
_B = 80
_NB = _N // _B
_NC = 2
_NS = 16
_NW = _NC * _NS
_BLK_LO = _NB // _NW
_EXTRA = _NB - _BLK_LO * _NW

_mesh = plsc.VectorSubcoreMesh(core_axis_name="c", subcore_axis_name="s")


def _lane_sums(accs):
    iota = lax.iota(jnp.int32, 16)
    vecs = list(accs)
    s = 1
    while len(vecs) > 1:
        perm = jnp.bitwise_xor(iota, s)
        mask = (iota & s) == 0
        nxt = []
        for i in range(0, len(vecs), 2):
            a, b = vecs[i], vecs[i + 1]
            d1 = jnp.where(mask, a, b)
            d2 = jnp.where(mask,
                           a.at[perm].get(mode="promise_in_bounds"),
                           b.at[perm].get(mode="promise_in_bounds"))
            nxt.append(d1 + d2)
        vecs = nxt
        s *= 2
    return vecs[0]


def _worker_range(wid):
    start = wid * _BLK_LO + jnp.minimum(wid, _EXTRA)
    cnt = _BLK_LO + jnp.where(wid < _EXTRA, 1, 0)
    return start, start + cnt


def _zero_rows(ref, nrows, ncols):
    z = jnp.zeros((16,), jnp.float32)
    for i in range(nrows):
        for j in range(ncols // 16):
            ref[i, pl.ds(16 * j, 16)] = z


@functools.partial(
    pl.kernel,
    out_type=(
        jax.ShapeDtypeStruct((2 * _G, _D), jnp.float32),
        jax.ShapeDtypeStruct((2 * _G, _D), jnp.float32),
    ),
    mesh=_mesh,
    scratch_types=[
        pltpu.VMEM((_B, _D), jnp.float32),
        pltpu.VMEM((_B,), jnp.int32),
        pltpu.VMEM((_B, _D), jnp.float32),
        pltpu.VMEM((16, _D), jnp.float32),
        pltpu.VMEM((16, _D), jnp.float32),
        pltpu.VMEM_SHARED((_G, _D), jnp.float32),
        pltpu.VMEM_SHARED((_G, _D), jnp.float32),
    ],
)
def _sc_pass1(x_hbm, b_hbm, sums_out, cnts_out, xb, bb, ones, tmp, tmpc, ssum, scnt):
    cid = lax.axis_index("c")
    sid = lax.axis_index("s")
    wid = cid * _NS + sid

    one = jnp.ones((16,), jnp.float32)
    for r in range(_B):
        for j in range(_D // 16):
            ones[r, pl.ds(16 * j, 16)] = one

    _zero_rows(tmp, 16, _D)
    _zero_rows(tmpc, 16, _D)
    pltpu.sync_copy(tmp, ssum.at[pl.ds(sid * 16, 16)])
    pltpu.sync_copy(tmpc, scnt.at[pl.ds(sid * 16, 16)])
    plsc.subcore_barrier()

    start, end = _worker_range(wid)

    def blk_body(blk, carry):
        base = pl.multiple_of(blk * _B, 16)
        pltpu.sync_copy(x_hbm.at[pl.ds(base, _B)], xb)
        pltpu.sync_copy(b_hbm.at[pl.ds(base, _B)], bb)
        pltpu.sync_copy(xb, ssum.at[bb], add=True)
        pltpu.sync_copy(ones, scnt.at[bb], add=True)
        return carry

    lax.fori_loop(start, end, blk_body, 0)
    plsc.subcore_barrier()

    pltpu.sync_copy(ssum.at[pl.ds(sid * 16, 16)], tmp)
    pltpu.sync_copy(tmp, sums_out.at[pl.ds(cid * _G + sid * 16, 16)])
    pltpu.sync_copy(scnt.at[pl.ds(sid * 16, 16)], tmpc)
    pltpu.sync_copy(tmpc, cnts_out.at[pl.ds(cid * _G + sid * 16, 16)])


@functools.partial(
    pl.kernel,
    out_type=jax.ShapeDtypeStruct((2 * _G, _D), jnp.float32),
    mesh=_mesh,
    scratch_types=[
        pltpu.VMEM((_B, _D), jnp.float32),
        pltpu.VMEM((_B,), jnp.int32),
        pltpu.VMEM((_B, _D), jnp.float32),
        pltpu.VMEM((16, _D), jnp.float32),
        pltpu.VMEM_SHARED((_G, _D), jnp.float32),
        pltpu.SemaphoreType.DMA,
    ],
)
def _sc_pass2(x_hbm, b_hbm, tg_hbm, out2, xb, bb, tgb, tmp, sout, gsem):
    cid = lax.axis_index("c")
    sid = lax.axis_index("s")
    wid = cid * _NS + sid

    _zero_rows(tmp, 16, _D)
    pltpu.sync_copy(tmp, sout.at[pl.ds(sid * 16, 16)])
    plsc.subcore_barrier()

    start, end = _worker_range(wid)
    lane_iota = lax.iota(jnp.int32, 16)

    def blk_body(blk, carry):
        base = pl.multiple_of(blk * _B, 16)
        pltpu.sync_copy(x_hbm.at[pl.ds(base, _B)], xb)
        pltpu.sync_copy(b_hbm.at[pl.ds(base, _B)], bb)
        pltpu.async_copy(tg_hbm.at[bb], tgb, gsem).wait()

        for g in range(_B // 16):
            rows = []
            accs = []
            for lane in range(16):
                r = 16 * g + lane
                xr = [xb[r, pl.ds(16 * j, 16)] for j in range(_D // 16)]
                rows.append(xr)
                acc = xr[0] * tgb[r, pl.ds(0, 16)]
                for j in range(1, _D // 16):
                    acc = acc + xr[j] * tgb[r, pl.ds(16 * j, 16)]
                accs.append(acc)
            dvec = _lane_sums(accs)
            cvec = 1.0 / (1.0 + jnp.exp(-dvec))
            for lane in range(16):
                r = 16 * g + lane
                c = cvec[lane]
                for j in range(_D // 16):
                    xb[r, pl.ds(16 * j, 16)] = rows[lane][j] * c

        pltpu.sync_copy(xb, sout.at[bb], add=True)
        return carry

    lax.fori_loop(start, end, blk_body, 0)
    plsc.subcore_barrier()

    pltpu.sync_copy(sout.at[pl.ds(sid * 16, 16)], tmp)
    pltpu.sync_copy(tmp, out2.at[pl.ds(cid * _G + sid * 16, 16)])


def _tc_mid(sums2, cnts2, W):
    def body(s_ref, c_ref, w_ref, o_ref):
        s = s_ref[pl.ds(0, _G), :] + s_ref[pl.ds(_G, _G), :]
        c = c_ref[pl.ds(0, _G), :][:, 0:1] + c_ref[pl.ds(_G, _G), :][:, 0:1]
        mean = s / jnp.maximum(c, 1.0)
        o_ref[...] = jnp.tanh(
            jnp.dot(mean, w_ref[...], preferred_element_type=jnp.float32))

    return pl.pallas_call(
        body, out_shape=jax.ShapeDtypeStruct((_G, _D), jnp.float32))(sums2, cnts2, W)


def _tc_combine(p2):
    def body(p_ref, o_ref):
        o_ref[...] = p_ref[pl.ds(0, _G), :] + p_ref[pl.ds(_G, _G), :]

    return pl.pallas_call(
        body, out_shape=jax.ShapeDtypeStruct((_G, _D), jnp.float32))(p2)


def kernel(x, batch, W):
    b32 = batch.astype(jnp.int32)
    sums2, cnts2 = _sc_pass1(x, b32)
    tg = _tc_mid(sums2, cnts2, W)
    p2 = _sc_pass2(x, b32, tg)
    return _tc_combine(p2)


# speedup vs baseline: 2.3959x; 1.4749x over previous
import functools

import jax
import jax.numpy as jnp
from jax import lax
from jax.experimental import pallas as pl
from jax.experimental.pallas import tpu as pltpu
from jax.experimental.pallas import tpu_sc as plsc

_N = 100000
_D = 128
_G = 256
_B = 80
_NB = _N // _B
_NC = 2
_NS = 16
_NW = _NC * _NS
_BLK_LO = _NB // _NW
_EXTRA = _NB - _BLK_LO * _NW

_mesh = plsc.VectorSubcoreMesh(core_axis_name="c", subcore_axis_name="s")


def _lane_sums(accs):
    iota = lax.iota(jnp.int32, 16)
    vecs = list(accs)
    s = 1
    while len(vecs) > 1:
        perm = jnp.bitwise_xor(iota, s)
        mask = (iota & s) == 0
        nxt = []
        for i in range(0, len(vecs), 2):
            a, b = vecs[i], vecs[i + 1]
            d1 = jnp.where(mask, a, b)
            d2 = jnp.where(mask,
                           a.at[perm].get(mode="promise_in_bounds"),
                           b.at[perm].get(mode="promise_in_bounds"))
            nxt.append(d1 + d2)
        vecs = nxt
        s *= 2
    return vecs[0]


def _worker_range(wid):
    start = wid * _BLK_LO + jnp.minimum(wid, _EXTRA)
    cnt = _BLK_LO + jnp.where(wid < _EXTRA, 1, 0)
    return start, start + cnt


def _zero_rows(ref, nrows, ncols):
    z = jnp.zeros((16,), jnp.float32)
    for i in range(nrows):
        for j in range(ncols // 16):
            ref[i, pl.ds(16 * j, 16)] = z


@functools.partial(
    pl.kernel,
    out_type=(
        jax.ShapeDtypeStruct((2 * _G, _D), jnp.float32),
        jax.ShapeDtypeStruct((2 * _G, _D), jnp.float32),
    ),
    mesh=_mesh,
    scratch_types=[
        pltpu.VMEM((_B, _D), jnp.float32),
        pltpu.VMEM((_B,), jnp.int32),
        pltpu.VMEM((_B, _D), jnp.float32),
        pltpu.VMEM((16, _D), jnp.float32),
        pltpu.VMEM((16, _D), jnp.float32),
        pltpu.VMEM_SHARED((_G, _D), jnp.float32),
        pltpu.VMEM_SHARED((_G, _D), jnp.float32),
    ],
)
def _sc_pass1(x_hbm, b_hbm, sums_out, cnts_out, xb, bb, ones, tmp, tmpc, ssum, scnt):
    cid = lax.axis_index("c")
    sid = lax.axis_index("s")
    wid = cid * _NS + sid

    one = jnp.ones((16,), jnp.float32)
    for r in range(_B):
        for j in range(_D // 16):
            ones[r, pl.ds(16 * j, 16)] = one

    _zero_rows(tmp, 16, _D)
    _zero_rows(tmpc, 16, _D)
    pltpu.sync_copy(tmp, ssum.at[pl.ds(sid * 16, 16)])
    pltpu.sync_copy(tmpc, scnt.at[pl.ds(sid * 16, 16)])
    plsc.subcore_barrier()

    start, end = _worker_range(wid)

    def blk_body(blk, carry):
        base = pl.multiple_of(blk * _B, 16)
        pltpu.sync_copy(x_hbm.at[pl.ds(base, _B)], xb)
        pltpu.sync_copy(b_hbm.at[pl.ds(base, _B)], bb)
        pltpu.sync_copy(xb, ssum.at[bb], add=True)
        pltpu.sync_copy(ones, scnt.at[bb], add=True)
        return carry

    lax.fori_loop(start, end, blk_body, 0)
    plsc.subcore_barrier()

    pltpu.sync_copy(ssum.at[pl.ds(sid * 16, 16)], tmp)
    pltpu.sync_copy(tmp, sums_out.at[pl.ds(cid * _G + sid * 16, 16)])
    pltpu.sync_copy(scnt.at[pl.ds(sid * 16, 16)], tmpc)
    pltpu.sync_copy(tmpc, cnts_out.at[pl.ds(cid * _G + sid * 16, 16)])


@functools.partial(
    pl.kernel,
    out_type=jax.ShapeDtypeStruct((2 * _G, _D), jnp.float32),
    mesh=_mesh,
    scratch_types=[
        pltpu.VMEM((_B, _D), jnp.float32),
        pltpu.VMEM((_B,), jnp.int32),
        pltpu.VMEM((_B, _D), jnp.float32),
        pltpu.VMEM((16, _D), jnp.float32),
        pltpu.VMEM_SHARED((_G, _D), jnp.float32),
        pltpu.VMEM_SHARED((_G, _D), jnp.float32),
        pltpu.SemaphoreType.DMA,
    ],
)
def _sc_pass2(x_hbm, b_hbm, tg_hbm, out2, xb, bb, tgb, tmp, stg, sout, gsem):
    cid = lax.axis_index("c")
    sid = lax.axis_index("s")
    wid = cid * _NS + sid

    pltpu.sync_copy(tg_hbm.at[pl.ds(sid * 16, 16)], stg.at[pl.ds(sid * 16, 16)])
    _zero_rows(tmp, 16, _D)
    pltpu.sync_copy(tmp, sout.at[pl.ds(sid * 16, 16)])
    plsc.subcore_barrier()

    start, end = _worker_range(wid)

    def blk_body(blk, carry):
        base = pl.multiple_of(blk * _B, 16)
        pltpu.sync_copy(x_hbm.at[pl.ds(base, _B)], xb)
        pltpu.sync_copy(b_hbm.at[pl.ds(base, _B)], bb)
        pltpu.async_copy(stg.at[bb], tgb, gsem).wait()

        for g in range(_B // 16):
            rows = []
            accs = []
            for lane in range(16):
                r = 16 * g + lane
                xr = [xb[r, pl.ds(16 * j, 16)] for j in range(_D // 16)]
                rows.append(xr)
                acc = xr[0] * tgb[r, pl.ds(0, 16)]
                for j in range(1, _D // 16):
                    acc = acc + xr[j] * tgb[r, pl.ds(16 * j, 16)]
                accs.append(acc)
            dvec = _lane_sums(accs)
            cvec = 1.0 / (1.0 + jnp.exp(-dvec))
            for lane in range(16):
                r = 16 * g + lane
                c = cvec[lane]
                for j in range(_D // 16):
                    xb[r, pl.ds(16 * j, 16)] = rows[lane][j] * c

        pltpu.sync_copy(xb, sout.at[bb], add=True)
        return carry

    lax.fori_loop(start, end, blk_body, 0)
    plsc.subcore_barrier()

    pltpu.sync_copy(sout.at[pl.ds(sid * 16, 16)], tmp)
    pltpu.sync_copy(tmp, out2.at[pl.ds(cid * _G + sid * 16, 16)])


def _tc_mid(sums2, cnts2, W):
    def body(s_ref, c_ref, w_ref, o_ref):
        s = s_ref[pl.ds(0, _G), :] + s_ref[pl.ds(_G, _G), :]
        c = c_ref[pl.ds(0, _G), :][:, 0:1] + c_ref[pl.ds(_G, _G), :][:, 0:1]
        mean = s / jnp.maximum(c, 1.0)
        o_ref[...] = jnp.tanh(
            jnp.dot(mean, w_ref[...], preferred_element_type=jnp.float32))

    return pl.pallas_call(
        body, out_shape=jax.ShapeDtypeStruct((_G, _D), jnp.float32))(sums2, cnts2, W)


def _tc_combine(p2):
    def body(p_ref, o_ref):
        o_ref[...] = p_ref[pl.ds(0, _G), :] + p_ref[pl.ds(_G, _G), :]

    return pl.pallas_call(
        body, out_shape=jax.ShapeDtypeStruct((_G, _D), jnp.float32))(p2)


def kernel(x, batch, W):
    b32 = batch.astype(jnp.int32)
    sums2, cnts2 = _sc_pass1(x, b32)
    tg = _tc_mid(sums2, cnts2, W)
    p2 = _sc_pass2(x, b32, tg)
    return _tc_combine(p2)


# speedup vs baseline: 2.5359x; 1.0584x over previous
import functools

import jax
import jax.numpy as jnp
from jax import lax
from jax.experimental import pallas as pl
from jax.experimental.pallas import tpu as pltpu
from jax.experimental.pallas import tpu_sc as plsc

_N = 100000
_D = 128
_G = 256
_B = 80
_NB = _N // _B
_NC = 2
_NS = 16
_NW = _NC * _NS
_BLK_LO = _NB // _NW
_EXTRA = _NB - _BLK_LO * _NW

_mesh = plsc.VectorSubcoreMesh(core_axis_name="c", subcore_axis_name="s")


def _lane_sums(accs):
    iota = lax.iota(jnp.int32, 16)
    vecs = list(accs)
    s = 1
    while len(vecs) > 1:
        perm = jnp.bitwise_xor(iota, s)
        mask = (iota & s) == 0
        nxt = []
        for i in range(0, len(vecs), 2):
            a, b = vecs[i], vecs[i + 1]
            d1 = jnp.where(mask, a, b)
            d2 = jnp.where(mask,
                           a.at[perm].get(mode="promise_in_bounds"),
                           b.at[perm].get(mode="promise_in_bounds"))
            nxt.append(d1 + d2)
        vecs = nxt
        s *= 2
    return vecs[0]


def _worker_range(wid):
    start = wid * _BLK_LO + jnp.minimum(wid, _EXTRA)
    cnt = _BLK_LO + jnp.where(wid < _EXTRA, 1, 0)
    return start, start + cnt


def _zero_rows(ref, nrows, ncols):
    z = jnp.zeros((16,), jnp.float32)
    for i in range(nrows):
        for j in range(ncols // 16):
            ref[i, pl.ds(16 * j, 16)] = z


@functools.partial(
    pl.kernel,
    out_type=(
        jax.ShapeDtypeStruct((2 * _G, _D), jnp.float32),
        jax.ShapeDtypeStruct((2 * _G, _D), jnp.float32),
    ),
    mesh=_mesh,
    scratch_types=[
        pltpu.VMEM((_B, _D), jnp.float32),
        pltpu.VMEM((_B,), jnp.int32),
        pltpu.VMEM((_B, _D), jnp.float32),
        pltpu.VMEM((16, _D), jnp.float32),
        pltpu.VMEM((16, _D), jnp.float32),
        pltpu.VMEM_SHARED((_G, _D), jnp.float32),
        pltpu.VMEM_SHARED((_G, _D), jnp.float32),
    ],
)
def _sc_pass1(x_hbm, b_hbm, sums_out, cnts_out, xb, bb, ones, tmp, tmpc, ssum, scnt):
    cid = lax.axis_index("c")
    sid = lax.axis_index("s")
    wid = cid * _NS + sid

    one = jnp.ones((16,), jnp.float32)
    for r in range(_B):
        for j in range(_D // 16):
            ones[r, pl.ds(16 * j, 16)] = one

    _zero_rows(tmp, 16, _D)
    _zero_rows(tmpc, 16, _D)
    pltpu.sync_copy(tmp, ssum.at[pl.ds(sid * 16, 16)])
    pltpu.sync_copy(tmpc, scnt.at[pl.ds(sid * 16, 16)])
    plsc.subcore_barrier()

    start, end = _worker_range(wid)

    def blk_body(blk, carry):
        base = pl.multiple_of(blk * _B, 16)
        pltpu.sync_copy(x_hbm.at[pl.ds(base, _B)], xb)
        pltpu.sync_copy(b_hbm.at[pl.ds(base, _B)], bb)
        pltpu.sync_copy(xb, ssum.at[bb], add=True)
        pltpu.sync_copy(ones, scnt.at[bb], add=True)
        return carry

    lax.fori_loop(start, end, blk_body, 0)
    plsc.subcore_barrier()

    pltpu.sync_copy(ssum.at[pl.ds(sid * 16, 16)], tmp)
    pltpu.sync_copy(tmp, sums_out.at[pl.ds(cid * _G + sid * 16, 16)])
    pltpu.sync_copy(scnt.at[pl.ds(sid * 16, 16)], tmpc)
    pltpu.sync_copy(tmpc, cnts_out.at[pl.ds(cid * _G + sid * 16, 16)])


@functools.partial(
    pl.kernel,
    out_type=jax.ShapeDtypeStruct((2 * _G, _D), jnp.float32),
    mesh=_mesh,
    scratch_types=[
        pltpu.VMEM((_B, _D), jnp.float32),
        pltpu.VMEM((_B, _D), jnp.float32),
        pltpu.VMEM((_B,), jnp.int32),
        pltpu.VMEM((_B,), jnp.int32),
        pltpu.VMEM((_B, _D), jnp.float32),
        pltpu.VMEM((16, _D), jnp.float32),
        pltpu.VMEM_SHARED((_G, _D), jnp.float32),
        pltpu.VMEM_SHARED((_G, _D), jnp.float32),
        pltpu.SemaphoreType.DMA,
        pltpu.SemaphoreType.DMA,
        pltpu.SemaphoreType.DMA,
    ],
)
def _sc_pass2(x_hbm, b_hbm, tg_hbm, out2, xbA, xbB, bbA, bbB, tgb, tmp,
              stg, sout, gsem, semA, semB):
    cid = lax.axis_index("c")
    sid = lax.axis_index("s")
    wid = cid * _NS + sid

    pltpu.sync_copy(tg_hbm.at[pl.ds(sid * 16, 16)], stg.at[pl.ds(sid * 16, 16)])
    _zero_rows(tmp, 16, _D)
    pltpu.sync_copy(tmp, sout.at[pl.ds(sid * 16, 16)])
    plsc.subcore_barrier()

    start, end = _worker_range(wid)

    def _start_loads(blk, xb, bb, sem):
        base = pl.multiple_of(blk * _B, 16)
        pltpu.make_async_copy(x_hbm.at[pl.ds(base, _B)], xb, sem).start()
        pltpu.make_async_copy(b_hbm.at[pl.ds(base, _B)], bb, sem).start()

    def _wait_loads(xb, bb, sem):
        pltpu.make_async_copy(x_hbm.at[pl.ds(0, _B)], xb, sem).wait()
        pltpu.make_async_copy(b_hbm.at[pl.ds(0, _B)], bb, sem).wait()

    def _compute_block(xb, bb):
        pltpu.async_copy(stg.at[bb], tgb, gsem).wait()
        for g in range(_B // 16):
            rows = []
            accs = []
            for lane in range(16):
                r = 16 * g + lane
                xr = [xb[r, pl.ds(16 * j, 16)] for j in range(_D // 16)]
                rows.append(xr)
                acc = xr[0] * tgb[r, pl.ds(0, 16)]
                for j in range(1, _D // 16):
                    acc = acc + xr[j] * tgb[r, pl.ds(16 * j, 16)]
                accs.append(acc)
            dvec = _lane_sums(accs)
            cvec = 1.0 / (1.0 + jnp.exp(-dvec))
            for lane in range(16):
                r = 16 * g + lane
                c = cvec[lane]
                for j in range(_D // 16):
                    xb[r, pl.ds(16 * j, 16)] = rows[lane][j] * c
        pltpu.sync_copy(xb, sout.at[bb], add=True)

    _start_loads(start, xbA, bbA, semA)

    def pair_body(k, carry):
        b0 = start + 2 * k
        b1 = b0 + 1
        _wait_loads(xbA, bbA, semA)

        @pl.when(b1 < end)
        def _():
            _start_loads(b1, xbB, bbB, semB)

        _compute_block(xbA, bbA)

        @pl.when(b1 < end)
        def _():
            _wait_loads(xbB, bbB, semB)

            @pl.when(b1 + 1 < end)
            def _():
                _start_loads(b1 + 1, xbA, bbA, semA)

            _compute_block(xbB, bbB)
        return carry

    npairs = (end - start + 1) // 2
    lax.fori_loop(0, npairs, pair_body, 0)
    plsc.subcore_barrier()

    pltpu.sync_copy(sout.at[pl.ds(sid * 16, 16)], tmp)
    pltpu.sync_copy(tmp, out2.at[pl.ds(cid * _G + sid * 16, 16)])


def _tc_mid(sums2, cnts2, W):
    def body(s_ref, c_ref, w_ref, o_ref):
        s = s_ref[pl.ds(0, _G), :] + s_ref[pl.ds(_G, _G), :]
        c = c_ref[pl.ds(0, _G), :][:, 0:1] + c_ref[pl.ds(_G, _G), :][:, 0:1]
        mean = s / jnp.maximum(c, 1.0)
        o_ref[...] = jnp.tanh(
            jnp.dot(mean, w_ref[...], preferred_element_type=jnp.float32))

    return pl.pallas_call(
        body, out_shape=jax.ShapeDtypeStruct((_G, _D), jnp.float32))(sums2, cnts2, W)


def _tc_combine(p2):
    def body(p_ref, o_ref):
        o_ref[...] = p_ref[pl.ds(0, _G), :] + p_ref[pl.ds(_G, _G), :]

    return pl.pallas_call(
        body, out_shape=jax.ShapeDtypeStruct((_G, _D), jnp.float32))(p2)


def kernel(x, batch, W):
    b32 = batch.astype(jnp.int32)
    sums2, cnts2 = _sc_pass1(x, b32)
    tg = _tc_mid(sums2, cnts2, W)
    p2 = _sc_pass2(x, b32, tg)
    return _tc_combine(p2)
